# baseline (speedup 1.0000x reference)
import functools

import jax
import jax.numpy as jnp
from jax import lax
from jax.experimental import pallas as pl
from jax.experimental.pallas import tpu as pltpu
from jax.experimental.pallas import tpu_sc as plsc

N = 10000
E = 320000
D_IN = 128
D_HID = 16
D_OUT = 40

NC = 2
NS = 16
NW = NC * NS
CHUNK = 128
E_PAD = 327680
CHUNKS = E_PAD // CHUNK
CPT = CHUNKS // NW
PAD_IDX = N
N_PAD = 10016
ROWS_PT = N_PAD // NS
R_PK = N_PAD // 8
R_N = N // 8

G = 8
L = 4
SD = 4
assert CPT % G == 0 and CPT % SD == 0


@functools.cache
def _build_sc_kernels():
    mesh = plsc.VectorSubcoreMesh(
        core_axis_name="c", subcore_axis_name="s",
        num_cores=NC, num_subcores=NS)
    cp = pltpu.CompilerParams(use_tc_tiling_on_sc=False)

    @functools.partial(
        pl.kernel,
        out_type=jax.ShapeDtypeStruct((NC, N_PAD, D_HID), jnp.float32),
        mesh=mesh,
        scratch_types=[
            pltpu.VMEM((CPT, CHUNK), jnp.int32),
            pltpu.VMEM((CPT, CHUNK), jnp.int32),
            pltpu.VMEM((G, CHUNK, D_HID), jnp.float32),
            pltpu.VMEM_SHARED((N_PAD, D_HID), jnp.float32),
        ] + [pltpu.SemaphoreType.DMA] * (2 * G),
        compiler_params=cp,
    )
    def sc_agg(table_hbm, eidx_hbm, zeros_hbm, out_hbm,
               src_v, dst_v, rows_v, acc_sh, *sems):
        gsem = sems[:G]
        ssem = sems[G:]
        c = lax.axis_index("c")
        s = lax.axis_index("s")
        wid = s * NC + c
        sl = pl.ds(s * ROWS_PT, ROWS_PT)

        @pl.when(c == 0)
        def _():
            pltpu.sync_copy(table_hbm.at[sl], acc_sh.at[sl])

        @pl.when(c != 0)
        def _():
            pltpu.sync_copy(zeros_hbm, acc_sh.at[sl])

        base = wid * CPT
        pltpu.sync_copy(eidx_hbm.at[0].at[pl.ds(base, CPT)], src_v)
        pltpu.sync_copy(eidx_hbm.at[1].at[pl.ds(base, CPT)], dst_v)
        plsc.subcore_barrier()

        for b in range(L):
            pltpu.async_copy(table_hbm.at[src_v.at[b]], rows_v.at[b], gsem[b])

        @pl.loop(0, CPT, step=G)
        def _(j):
            for b in range(G):
                t = j + b
                pltpu.make_async_copy(
                    table_hbm.at[src_v.at[t]], rows_v.at[b], gsem[b]).wait()
                pltpu.async_copy(
                    rows_v.at[b], acc_sh.at[dst_v.at[t]], ssem[b], add=True)
                bb = (b + L) % G
                nxt = t + L

                @pl.when(nxt < CPT)
                def _():
                    @pl.when(nxt >= G)
                    def _():
                        pltpu.make_async_copy(
                            rows_v.at[bb], acc_sh.at[dst_v.at[t]],
                            ssem[bb]).wait()

                    pltpu.async_copy(
                        table_hbm.at[src_v.at[nxt]], rows_v.at[bb],
                        gsem[bb])

        for b in range(G):
            pltpu.make_async_copy(
                rows_v.at[b], acc_sh.at[dst_v.at[CPT - 1]], ssem[b]).wait()

        plsc.subcore_barrier()
        pltpu.sync_copy(acc_sh.at[sl], out_hbm.at[c].at[sl])

    @functools.partial(
        pl.kernel,
        out_type=jax.ShapeDtypeStruct((NC, N_PAD, D_HID), jnp.float32),
        mesh=mesh,
        scratch_types=[
            pltpu.VMEM((CPT, CHUNK), jnp.int32),
            pltpu.VMEM((CHUNK, D_HID), jnp.float32),
            pltpu.VMEM_SHARED((N_PAD, D_HID), jnp.float32),
        ] + [pltpu.SemaphoreType.DMA] * SD,
        compiler_params=cp,
    )
    def sc_deg(ones_hbm, eidx_hbm, zeros_hbm, out_hbm, dst_v, ones_v, acc_sh,
               *ssem):
        c = lax.axis_index("c")
        s = lax.axis_index("s")
        wid = s * NC + c
        sl = pl.ds(s * ROWS_PT, ROWS_PT)

        @pl.when(c == 0)
        def _():
            pltpu.sync_copy(ones_hbm, acc_sh.at[sl])

        @pl.when(c != 0)
        def _():
            pltpu.sync_copy(zeros_hbm, acc_sh.at[sl])

        pltpu.sync_copy(ones_hbm.at[pl.ds(0, CHUNK)], ones_v)
        base = wid * CPT
        pltpu.sync_copy(eidx_hbm.at[1].at[pl.ds(base, CPT)], dst_v)
        plsc.subcore_barrier()

        @pl.loop(0, CPT, step=SD)
        def _(j):
            for b in range(SD):
                t = j + b

                @pl.when(t >= SD)
                def _():
                    pltpu.make_async_copy(
                        ones_v, acc_sh.at[dst_v.at[t]], ssem[b]).wait()

                pltpu.async_copy(ones_v, acc_sh.at[dst_v.at[t]], ssem[b],
                                 add=True)

        for b in range(SD):
            pltpu.make_async_copy(
                ones_v, acc_sh.at[dst_v.at[CPT - 1]], ssem[b]).wait()

        plsc.subcore_barrier()
        pltpu.sync_copy(acc_sh.at[sl], out_hbm.at[c].at[sl])

    return sc_deg, sc_agg


def _mm1_body(x_ref, w_ref, o_ref):
    o_ref[...] = lax.dot_general(
        x_ref[...], w_ref[...], (((1,), (0,)), ((), ())),
        preferred_element_type=jnp.float32)


def _scale_body(degp_ref, t1_ref, t1s_ref, dinv_ref):
    deg = degp_ref[0] + degp_ref[1]
    dinv = lax.rsqrt(jnp.maximum(deg, 1e-12))
    dinv_ref[...] = dinv
    t1s_ref[...] = t1_ref[...] * dinv


def _mid_body(aggp_ref, dinv_ref, b1t_ref, t2s_ref):
    agg = aggp_ref[0] + aggp_ref[1]
    dinv = dinv_ref[...]
    h = jnp.maximum(agg * dinv + b1t_ref[...], 0.0)
    t2s_ref[...] = h * dinv


def _out_body(aggp_ref, dinv_ref, w2_ref, b2_ref, o_ref):
    g = ((aggp_ref[0] + aggp_ref[1]) * dinv_ref[...])[:N]
    z = lax.dot_general(
        g, w2_ref[...], (((1,), (1,)), ((), ())),
        preferred_element_type=jnp.float32) + b2_ref[...]
    m = jnp.max(z, axis=1, keepdims=True)
    lse = jnp.log(jnp.sum(jnp.exp(z - m), axis=1, keepdims=True)) + m
    o_ref[...] = z - lse


_mm1 = pl.pallas_call(
    _mm1_body, out_shape=jax.ShapeDtypeStruct((R_PK, 128), jnp.float32))
_scale = pl.pallas_call(
    _scale_body,
    out_shape=(jax.ShapeDtypeStruct((R_PK, 128), jnp.float32),
               jax.ShapeDtypeStruct((R_PK, 128), jnp.float32)))
_mid = pl.pallas_call(
    _mid_body, out_shape=jax.ShapeDtypeStruct((R_PK, 128), jnp.float32))
_out = pl.pallas_call(
    _out_body, out_shape=jax.ShapeDtypeStruct((N, D_OUT), jnp.float32))


def _pk(a):
    return jnp.reshape(a, (-1, R_PK, 128)) if a.ndim == 3 else (
        jnp.reshape(a, (R_PK, 128)))


def kernel(x, edge_index, W1, b1, W2, b2):
    eidx = jnp.pad(edge_index, ((0, 0), (0, E_PAD - E)),
                   constant_values=PAD_IDX).reshape(2, CHUNKS, CHUNK)
    x_pk = jnp.pad(x, ((0, N_PAD - N), (0, 0))).reshape(R_PK, 8 * D_IN)
    w1_pk = jnp.kron(jnp.eye(8, dtype=jnp.float32), W1.T)
    b1t = jnp.tile(b1, 8).reshape(1, 128)
    zeros_sm = jnp.zeros((ROWS_PT, D_HID), jnp.float32)
    ones_sm = jnp.ones((ROWS_PT, D_HID), jnp.float32)

    sc_deg, sc_agg = _build_sc_kernels()
    degp = sc_deg(ones_sm, eidx, zeros_sm)
    t1_pk = _mm1(x_pk, w1_pk)
    t1s_pk, dinv_pk = _scale(_pk(degp), t1_pk)
    t1s = jnp.reshape(t1s_pk, (N_PAD, D_HID))
    aggp1 = sc_agg(t1s, eidx, zeros_sm)
    t2s_pk = _mid(_pk(aggp1), dinv_pk, b1t)
    t2s = jnp.reshape(t2s_pk, (N_PAD, D_HID))
    aggp2 = sc_agg(t2s, eidx, zeros_sm)
    dinv_un = jnp.reshape(dinv_pk, (N_PAD, D_HID))
    return _out(aggp2, dinv_un, W2, b2.reshape(1, D_OUT))


# speedup vs baseline: 1.0469x; 1.0469x over previous
import functools

import jax
import jax.numpy as jnp
from jax import lax
from jax.experimental import pallas as pl
from jax.experimental.pallas import tpu as pltpu
from jax.experimental.pallas import tpu_sc as plsc

N = 10000
E = 320000
D_IN = 128
D_HID = 16
D_OUT = 40

NC = 2
NS = 16
NW = NC * NS
CHUNK = 128
E_PAD = 327680
CHUNKS = E_PAD // CHUNK
CPT = CHUNKS // NW
PAD_IDX = N
N_PAD = 10016
ROWS_PT = N_PAD // NS
R_PK = N_PAD // 8
R_N = N // 8

G = 8
L = 4
SD = 4
assert CPT % G == 0 and CPT % SD == 0

CPT0 = 120
CPT1 = 160 - CPT0
assert CPT0 % G == 0 and CPT1 % G == 0
CPT_MAX = max(CPT0, CPT1)


@functools.cache
def _build_sc_kernels():
    mesh = plsc.VectorSubcoreMesh(
        core_axis_name="c", subcore_axis_name="s",
        num_cores=NC, num_subcores=NS)
    cp = pltpu.CompilerParams(use_tc_tiling_on_sc=False)

    @functools.partial(
        pl.kernel,
        out_type=jax.ShapeDtypeStruct((NC, N_PAD, D_HID), jnp.float32),
        mesh=mesh,
        scratch_types=[
            pltpu.VMEM((CPT_MAX, CHUNK), jnp.int32),
            pltpu.VMEM((CPT_MAX, CHUNK), jnp.int32),
            pltpu.VMEM((G, CHUNK, D_HID), jnp.float32),
            pltpu.VMEM_SHARED((N_PAD, D_HID), jnp.float32),
        ] + [pltpu.SemaphoreType.DMA] * (2 * G),
        compiler_params=cp,
    )
    def sc_agg(table_hbm, eidx_hbm, zeros_hbm, out_hbm,
               src_v, dst_v, rows_v, acc_sh, *sems):
        gsem = sems[:G]
        ssem = sems[G:]
        c = lax.axis_index("c")
        s = lax.axis_index("s")
        sl = pl.ds(s * ROWS_PT, ROWS_PT)

        @pl.when(c == 0)
        def _():
            pltpu.sync_copy(table_hbm.at[sl], acc_sh.at[sl])

        @pl.when(c != 0)
        def _():
            pltpu.sync_copy(zeros_hbm, acc_sh.at[sl])

        def run(cpt, base):
            pltpu.sync_copy(eidx_hbm.at[0].at[pl.ds(base, cpt)], src_v.at[pl.ds(0, cpt)])
            pltpu.sync_copy(eidx_hbm.at[1].at[pl.ds(base, cpt)], dst_v.at[pl.ds(0, cpt)])
            plsc.subcore_barrier()

            for b in range(L):
                pltpu.async_copy(
                    table_hbm.at[src_v.at[b]], rows_v.at[b], gsem[b])

            @pl.loop(0, cpt, step=G)
            def _(j):
                for b in range(G):
                    t = j + b
                    pltpu.make_async_copy(
                        table_hbm.at[src_v.at[t]], rows_v.at[b],
                        gsem[b]).wait()
                    pltpu.async_copy(
                        rows_v.at[b], acc_sh.at[dst_v.at[t]], ssem[b],
                        add=True)
                    bb = (b + L) % G
                    nxt = t + L

                    @pl.when(nxt < cpt)
                    def _():
                        @pl.when(nxt >= G)
                        def _():
                            pltpu.make_async_copy(
                                rows_v.at[bb], acc_sh.at[dst_v.at[t]],
                                ssem[bb]).wait()

                        pltpu.async_copy(
                            table_hbm.at[src_v.at[nxt]], rows_v.at[bb],
                            gsem[bb])

            for b in range(G):
                pltpu.make_async_copy(
                    rows_v.at[b], acc_sh.at[dst_v.at[cpt - 1]],
                    ssem[b]).wait()

        @pl.when(c == 0)
        def _():
            run(CPT0, s * CPT0)

        @pl.when(c != 0)
        def _():
            run(CPT1, NS * CPT0 + s * CPT1)

        plsc.subcore_barrier()
        pltpu.sync_copy(acc_sh.at[sl], out_hbm.at[c].at[sl])

    @functools.partial(
        pl.kernel,
        out_type=jax.ShapeDtypeStruct((NC, N_PAD, D_HID), jnp.float32),
        mesh=mesh,
        scratch_types=[
            pltpu.VMEM((CPT, CHUNK), jnp.int32),
            pltpu.VMEM((CHUNK, D_HID), jnp.float32),
            pltpu.VMEM_SHARED((N_PAD, D_HID), jnp.float32),
        ] + [pltpu.SemaphoreType.DMA] * SD,
        compiler_params=cp,
    )
    def sc_deg(ones_hbm, eidx_hbm, zeros_hbm, out_hbm, dst_v, ones_v, acc_sh,
               *ssem):
        c = lax.axis_index("c")
        s = lax.axis_index("s")
        wid = s * NC + c
        sl = pl.ds(s * ROWS_PT, ROWS_PT)

        @pl.when(c == 0)
        def _():
            pltpu.sync_copy(ones_hbm, acc_sh.at[sl])

        @pl.when(c != 0)
        def _():
            pltpu.sync_copy(zeros_hbm, acc_sh.at[sl])

        pltpu.sync_copy(ones_hbm.at[pl.ds(0, CHUNK)], ones_v)
        base = wid * CPT
        pltpu.sync_copy(eidx_hbm.at[1].at[pl.ds(base, CPT)], dst_v)
        plsc.subcore_barrier()

        @pl.loop(0, CPT, step=SD)
        def _(j):
            for b in range(SD):
                t = j + b

                @pl.when(t >= SD)
                def _():
                    pltpu.make_async_copy(
                        ones_v, acc_sh.at[dst_v.at[t]], ssem[b]).wait()

                pltpu.async_copy(ones_v, acc_sh.at[dst_v.at[t]], ssem[b],
                                 add=True)

        for b in range(SD):
            pltpu.make_async_copy(
                ones_v, acc_sh.at[dst_v.at[CPT - 1]], ssem[b]).wait()

        plsc.subcore_barrier()
        pltpu.sync_copy(acc_sh.at[sl], out_hbm.at[c].at[sl])

    return sc_deg, sc_agg


def _mm1_body(x_ref, w_ref, o_ref):
    o_ref[...] = lax.dot_general(
        x_ref[...], w_ref[...], (((1,), (0,)), ((), ())),
        preferred_element_type=jnp.float32)


def _scale_body(degp_ref, t1_ref, t1s_ref, dinv_ref):
    deg = degp_ref[0] + degp_ref[1]
    dinv = lax.rsqrt(jnp.maximum(deg, 1e-12))
    dinv_ref[...] = dinv
    t1s_ref[...] = t1_ref[...] * dinv


def _mid_body(aggp_ref, dinv_ref, b1t_ref, t2s_ref):
    agg = aggp_ref[0] + aggp_ref[1]
    dinv = dinv_ref[...]
    h = jnp.maximum(agg * dinv + b1t_ref[...], 0.0)
    t2s_ref[...] = h * dinv


def _out_body(aggp_ref, dinv_ref, w2_ref, b2_ref, o_ref):
    g = ((aggp_ref[0] + aggp_ref[1]) * dinv_ref[...])[:N]
    z = lax.dot_general(
        g, w2_ref[...], (((1,), (1,)), ((), ())),
        preferred_element_type=jnp.float32) + b2_ref[...]
    m = jnp.max(z, axis=1, keepdims=True)
    lse = jnp.log(jnp.sum(jnp.exp(z - m), axis=1, keepdims=True)) + m
    o_ref[...] = z - lse


_mm1 = pl.pallas_call(
    _mm1_body, out_shape=jax.ShapeDtypeStruct((R_PK, 128), jnp.float32))
_scale = pl.pallas_call(
    _scale_body,
    out_shape=(jax.ShapeDtypeStruct((R_PK, 128), jnp.float32),
               jax.ShapeDtypeStruct((R_PK, 128), jnp.float32)))
_mid = pl.pallas_call(
    _mid_body, out_shape=jax.ShapeDtypeStruct((R_PK, 128), jnp.float32))
_out = pl.pallas_call(
    _out_body, out_shape=jax.ShapeDtypeStruct((N, D_OUT), jnp.float32))


def _pk(a):
    return jnp.reshape(a, (-1, R_PK, 128)) if a.ndim == 3 else (
        jnp.reshape(a, (R_PK, 128)))


def kernel(x, edge_index, W1, b1, W2, b2):
    eidx = jnp.pad(edge_index, ((0, 0), (0, E_PAD - E)),
                   constant_values=PAD_IDX).reshape(2, CHUNKS, CHUNK)
    x_pk = jnp.pad(x, ((0, N_PAD - N), (0, 0))).reshape(R_PK, 8 * D_IN)
    w1_pk = jnp.kron(jnp.eye(8, dtype=jnp.float32), W1.T)
    b1t = jnp.tile(b1, 8).reshape(1, 128)
    zeros_sm = jnp.zeros((ROWS_PT, D_HID), jnp.float32)
    ones_sm = jnp.ones((ROWS_PT, D_HID), jnp.float32)

    sc_deg, sc_agg = _build_sc_kernels()
    degp = sc_deg(ones_sm, eidx, zeros_sm)
    t1_pk = _mm1(x_pk, w1_pk)
    t1s_pk, dinv_pk = _scale(_pk(degp), t1_pk)
    t1s = jnp.reshape(t1s_pk, (N_PAD, D_HID))
    aggp1 = sc_agg(t1s, eidx, zeros_sm)
    t2s_pk = _mid(_pk(aggp1), dinv_pk, b1t)
    t2s = jnp.reshape(t2s_pk, (N_PAD, D_HID))
    aggp2 = sc_agg(t2s, eidx, zeros_sm)
    dinv_un = jnp.reshape(dinv_pk, (N_PAD, D_HID))
    return _out(aggp2, dinv_un, W2, b2.reshape(1, D_OUT))


# speedup vs baseline: 1.7632x; 1.6842x over previous
import functools

import jax
import jax.numpy as jnp
from jax import lax
from jax.experimental import pallas as pl
from jax.experimental.pallas import tpu as pltpu
from jax.experimental.pallas import tpu_sc as plsc

N = 10000
E = 320000
D_IN = 128
D_HID = 16
D_OUT = 40

NC = 2
NS = 16
NW = NC * NS
CHUNK = 128
E_PAD = 327680
CHUNKS = E_PAD // CHUNK
CPT = CHUNKS // NW
PAD_IDX = N
N_PAD = 10016
ROWS_PT = N_PAD // NS
R_PK = N_PAD // 8
R_N = N // 8

G = 8
L = 4
SD = 4
assert CPT % G == 0 and CPT % SD == 0

CPT0 = 80
CPT1 = 160 - CPT0
assert CPT0 % G == 0 and CPT1 % G == 0
CPT_MAX = max(CPT0, CPT1)


@functools.cache
def _build_sc_kernels():
    mesh = plsc.VectorSubcoreMesh(
        core_axis_name="c", subcore_axis_name="s",
        num_cores=NC, num_subcores=NS)
    cp = pltpu.CompilerParams(use_tc_tiling_on_sc=False)

    @functools.partial(
        pl.kernel,
        out_type=jax.ShapeDtypeStruct((NC, N_PAD, D_HID), jnp.float32),
        mesh=mesh,
        scratch_types=[
            pltpu.VMEM((CPT_MAX, CHUNK), jnp.int32),
            pltpu.VMEM((CPT_MAX, CHUNK), jnp.int32),
            pltpu.VMEM((G, CHUNK, D_HID), jnp.float32),
            pltpu.VMEM_SHARED((N_PAD, D_HID), jnp.float32),
            pltpu.VMEM_SHARED((N_PAD, D_HID), jnp.float32),
        ] + [pltpu.SemaphoreType.DMA] * (2 * G),
        compiler_params=cp,
    )
    def sc_agg(table_hbm, eidx_hbm, zeros_hbm, out_hbm,
               src_v, dst_v, rows_v, acc_sh, table_sh, *sems):
        gsem = sems[:G]
        ssem = sems[G:]
        c = lax.axis_index("c")
        s = lax.axis_index("s")
        sl = pl.ds(s * ROWS_PT, ROWS_PT)

        pltpu.sync_copy(table_hbm.at[sl], table_sh.at[sl])

        @pl.when(c == 0)
        def _():
            pltpu.sync_copy(table_hbm.at[sl], acc_sh.at[sl])

        @pl.when(c != 0)
        def _():
            pltpu.sync_copy(zeros_hbm, acc_sh.at[sl])

        def run(cpt, base):
            pltpu.sync_copy(eidx_hbm.at[0].at[pl.ds(base, cpt)], src_v.at[pl.ds(0, cpt)])
            pltpu.sync_copy(eidx_hbm.at[1].at[pl.ds(base, cpt)], dst_v.at[pl.ds(0, cpt)])
            plsc.subcore_barrier()

            for b in range(L):
                pltpu.async_copy(
                    table_sh.at[src_v.at[b]], rows_v.at[b], gsem[b])

            @pl.loop(0, cpt, step=G)
            def _(j):
                for b in range(G):
                    t = j + b
                    pltpu.make_async_copy(
                        table_sh.at[src_v.at[t]], rows_v.at[b],
                        gsem[b]).wait()
                    pltpu.async_copy(
                        rows_v.at[b], acc_sh.at[dst_v.at[t]], ssem[b],
                        add=True)
                    bb = (b + L) % G
                    nxt = t + L

                    @pl.when(nxt < cpt)
                    def _():
                        @pl.when(nxt >= G)
                        def _():
                            pltpu.make_async_copy(
                                rows_v.at[bb], acc_sh.at[dst_v.at[t]],
                                ssem[bb]).wait()

                        pltpu.async_copy(
                            table_sh.at[src_v.at[nxt]], rows_v.at[bb],
                            gsem[bb])

            for b in range(G):
                pltpu.make_async_copy(
                    rows_v.at[b], acc_sh.at[dst_v.at[cpt - 1]],
                    ssem[b]).wait()

        @pl.when(c == 0)
        def _():
            run(CPT0, s * CPT0)

        @pl.when(c != 0)
        def _():
            run(CPT1, NS * CPT0 + s * CPT1)

        plsc.subcore_barrier()
        pltpu.sync_copy(acc_sh.at[sl], out_hbm.at[c].at[sl])

    @functools.partial(
        pl.kernel,
        out_type=jax.ShapeDtypeStruct((NC, N_PAD, D_HID), jnp.float32),
        mesh=mesh,
        scratch_types=[
            pltpu.VMEM((CPT, CHUNK), jnp.int32),
            pltpu.VMEM((CHUNK, D_HID), jnp.float32),
            pltpu.VMEM_SHARED((N_PAD, D_HID), jnp.float32),
        ] + [pltpu.SemaphoreType.DMA] * SD,
        compiler_params=cp,
    )
    def sc_deg(ones_hbm, eidx_hbm, zeros_hbm, out_hbm, dst_v, ones_v, acc_sh,
               *ssem):
        c = lax.axis_index("c")
        s = lax.axis_index("s")
        wid = s * NC + c
        sl = pl.ds(s * ROWS_PT, ROWS_PT)

        @pl.when(c == 0)
        def _():
            pltpu.sync_copy(ones_hbm, acc_sh.at[sl])

        @pl.when(c != 0)
        def _():
            pltpu.sync_copy(zeros_hbm, acc_sh.at[sl])

        pltpu.sync_copy(ones_hbm.at[pl.ds(0, CHUNK)], ones_v)
        base = wid * CPT
        pltpu.sync_copy(eidx_hbm.at[1].at[pl.ds(base, CPT)], dst_v)
        plsc.subcore_barrier()

        @pl.loop(0, CPT, step=SD)
        def _(j):
            for b in range(SD):
                t = j + b

                @pl.when(t >= SD)
                def _():
                    pltpu.make_async_copy(
                        ones_v, acc_sh.at[dst_v.at[t]], ssem[b]).wait()

                pltpu.async_copy(ones_v, acc_sh.at[dst_v.at[t]], ssem[b],
                                 add=True)

        for b in range(SD):
            pltpu.make_async_copy(
                ones_v, acc_sh.at[dst_v.at[CPT - 1]], ssem[b]).wait()

        plsc.subcore_barrier()
        pltpu.sync_copy(acc_sh.at[sl], out_hbm.at[c].at[sl])

    return sc_deg, sc_agg


def _mm1_body(x_ref, w_ref, o_ref):
    o_ref[...] = lax.dot_general(
        x_ref[...], w_ref[...], (((1,), (0,)), ((), ())),
        preferred_element_type=jnp.float32)


def _scale_body(degp_ref, t1_ref, t1s_ref, dinv_ref):
    deg = degp_ref[0] + degp_ref[1]
    dinv = lax.rsqrt(jnp.maximum(deg, 1e-12))
    dinv_ref[...] = dinv
    t1s_ref[...] = t1_ref[...] * dinv


def _mid_body(aggp_ref, dinv_ref, b1t_ref, t2s_ref):
    agg = aggp_ref[0] + aggp_ref[1]
    dinv = dinv_ref[...]
    h = jnp.maximum(agg * dinv + b1t_ref[...], 0.0)
    t2s_ref[...] = h * dinv


def _out_body(aggp_ref, dinv_ref, w2_ref, b2_ref, o_ref):
    g = ((aggp_ref[0] + aggp_ref[1]) * dinv_ref[...])[:N]
    z = lax.dot_general(
        g, w2_ref[...], (((1,), (1,)), ((), ())),
        preferred_element_type=jnp.float32) + b2_ref[...]
    m = jnp.max(z, axis=1, keepdims=True)
    lse = jnp.log(jnp.sum(jnp.exp(z - m), axis=1, keepdims=True)) + m
    o_ref[...] = z - lse


_mm1 = pl.pallas_call(
    _mm1_body, out_shape=jax.ShapeDtypeStruct((R_PK, 128), jnp.float32))
_scale = pl.pallas_call(
    _scale_body,
    out_shape=(jax.ShapeDtypeStruct((R_PK, 128), jnp.float32),
               jax.ShapeDtypeStruct((R_PK, 128), jnp.float32)))
_mid = pl.pallas_call(
    _mid_body, out_shape=jax.ShapeDtypeStruct((R_PK, 128), jnp.float32))
_out = pl.pallas_call(
    _out_body, out_shape=jax.ShapeDtypeStruct((N, D_OUT), jnp.float32))


def _pk(a):
    return jnp.reshape(a, (-1, R_PK, 128)) if a.ndim == 3 else (
        jnp.reshape(a, (R_PK, 128)))


def kernel(x, edge_index, W1, b1, W2, b2):
    eidx = jnp.pad(edge_index, ((0, 0), (0, E_PAD - E)),
                   constant_values=PAD_IDX).reshape(2, CHUNKS, CHUNK)
    x_pk = jnp.pad(x, ((0, N_PAD - N), (0, 0))).reshape(R_PK, 8 * D_IN)
    w1_pk = jnp.kron(jnp.eye(8, dtype=jnp.float32), W1.T)
    b1t = jnp.tile(b1, 8).reshape(1, 128)
    zeros_sm = jnp.zeros((ROWS_PT, D_HID), jnp.float32)
    ones_sm = jnp.ones((ROWS_PT, D_HID), jnp.float32)

    sc_deg, sc_agg = _build_sc_kernels()
    degp = sc_deg(ones_sm, eidx, zeros_sm)
    t1_pk = _mm1(x_pk, w1_pk)
    t1s_pk, dinv_pk = _scale(_pk(degp), t1_pk)
    t1s = jnp.reshape(t1s_pk, (N_PAD, D_HID))
    aggp1 = sc_agg(t1s, eidx, zeros_sm)
    t2s_pk = _mid(_pk(aggp1), dinv_pk, b1t)
    t2s = jnp.reshape(t2s_pk, (N_PAD, D_HID))
    aggp2 = sc_agg(t2s, eidx, zeros_sm)
    dinv_un = jnp.reshape(dinv_pk, (N_PAD, D_HID))
    return _out(aggp2, dinv_un, W2, b2.reshape(1, D_OUT))


# speedup vs baseline: 1.8841x; 1.0686x over previous
import functools

import jax
import jax.numpy as jnp
from jax import lax
from jax.experimental import pallas as pl
from jax.experimental.pallas import tpu as pltpu
from jax.experimental.pallas import tpu_sc as plsc

N = 10000
E = 320000
D_IN = 128
D_HID = 16
D_OUT = 40

NC = 2
NS = 16
NW = NC * NS
CHUNK = 128
E_PAD = 327680
CHUNKS = E_PAD // CHUNK
CPT = CHUNKS // NW
PAD_IDX = N
N_PAD = 10016
ROWS_PT = N_PAD // NS
R_PK = N_PAD // 8
R_N = N // 8

G = 10
L = 5
SD = 8
assert CPT % G == 0 and CPT % SD == 0

CPT0 = 80
CPT1 = 160 - CPT0
assert CPT0 % G == 0 and CPT1 % G == 0
CPT_MAX = max(CPT0, CPT1)


@functools.cache
def _build_sc_kernels():
    mesh = plsc.VectorSubcoreMesh(
        core_axis_name="c", subcore_axis_name="s",
        num_cores=NC, num_subcores=NS)
    cp = pltpu.CompilerParams(use_tc_tiling_on_sc=False)

    @functools.partial(
        pl.kernel,
        out_type=jax.ShapeDtypeStruct((NC, N_PAD, D_HID), jnp.float32),
        mesh=mesh,
        scratch_types=[
            pltpu.VMEM((CPT_MAX, CHUNK), jnp.int32),
            pltpu.VMEM((CPT_MAX, CHUNK), jnp.int32),
            pltpu.VMEM((G, CHUNK, D_HID), jnp.float32),
            pltpu.VMEM_SHARED((N_PAD, D_HID), jnp.float32),
            pltpu.VMEM_SHARED((N_PAD, D_HID), jnp.float32),
        ] + [pltpu.SemaphoreType.DMA] * (2 * G),
        compiler_params=cp,
    )
    def sc_agg(table_hbm, eidx_hbm, zeros_hbm, out_hbm,
               src_v, dst_v, rows_v, acc_sh, table_sh, *sems):
        gsem = sems[:G]
        ssem = sems[G:]
        c = lax.axis_index("c")
        s = lax.axis_index("s")
        sl = pl.ds(s * ROWS_PT, ROWS_PT)

        pltpu.sync_copy(table_hbm.at[sl], table_sh.at[sl])

        @pl.when(c == 0)
        def _():
            pltpu.sync_copy(table_hbm.at[sl], acc_sh.at[sl])

        @pl.when(c != 0)
        def _():
            pltpu.sync_copy(zeros_hbm, acc_sh.at[sl])

        def run(cpt, base):
            pltpu.sync_copy(eidx_hbm.at[0].at[pl.ds(base, cpt)], src_v.at[pl.ds(0, cpt)])
            pltpu.sync_copy(eidx_hbm.at[1].at[pl.ds(base, cpt)], dst_v.at[pl.ds(0, cpt)])
            plsc.subcore_barrier()

            for b in range(L):
                pltpu.async_copy(
                    table_sh.at[src_v.at[b]], rows_v.at[b], gsem[b])

            @pl.loop(0, cpt, step=G)
            def _(j):
                for b in range(G):
                    t = j + b
                    pltpu.make_async_copy(
                        table_sh.at[src_v.at[t]], rows_v.at[b],
                        gsem[b]).wait()
                    pltpu.async_copy(
                        rows_v.at[b], acc_sh.at[dst_v.at[t]], ssem[b],
                        add=True)
                    bb = (b + L) % G
                    nxt = t + L

                    @pl.when(nxt < cpt)
                    def _():
                        @pl.when(nxt >= G)
                        def _():
                            pltpu.make_async_copy(
                                rows_v.at[bb], acc_sh.at[dst_v.at[t]],
                                ssem[bb]).wait()

                        pltpu.async_copy(
                            table_sh.at[src_v.at[nxt]], rows_v.at[bb],
                            gsem[bb])

            for b in range(G):
                pltpu.make_async_copy(
                    rows_v.at[b], acc_sh.at[dst_v.at[cpt - 1]],
                    ssem[b]).wait()

        @pl.when(c == 0)
        def _():
            run(CPT0, s * CPT0)

        @pl.when(c != 0)
        def _():
            run(CPT1, NS * CPT0 + s * CPT1)

        plsc.subcore_barrier()
        pltpu.sync_copy(acc_sh.at[sl], out_hbm.at[c].at[sl])

    @functools.partial(
        pl.kernel,
        out_type=jax.ShapeDtypeStruct((NC, N_PAD, D_HID), jnp.float32),
        mesh=mesh,
        scratch_types=[
            pltpu.VMEM((CPT, CHUNK), jnp.int32),
            pltpu.VMEM((CHUNK, D_HID), jnp.float32),
            pltpu.VMEM_SHARED((N_PAD, D_HID), jnp.float32),
        ] + [pltpu.SemaphoreType.DMA] * SD,
        compiler_params=cp,
    )
    def sc_deg(ones_hbm, eidx_hbm, zeros_hbm, out_hbm, dst_v, ones_v, acc_sh,
               *ssem):
        c = lax.axis_index("c")
        s = lax.axis_index("s")
        wid = s * NC + c
        sl = pl.ds(s * ROWS_PT, ROWS_PT)

        @pl.when(c == 0)
        def _():
            pltpu.sync_copy(ones_hbm, acc_sh.at[sl])

        @pl.when(c != 0)
        def _():
            pltpu.sync_copy(zeros_hbm, acc_sh.at[sl])

        pltpu.sync_copy(ones_hbm.at[pl.ds(0, CHUNK)], ones_v)
        base = wid * CPT
        pltpu.sync_copy(eidx_hbm.at[1].at[pl.ds(base, CPT)], dst_v)
        plsc.subcore_barrier()

        @pl.loop(0, CPT, step=SD)
        def _(j):
            for b in range(SD):
                t = j + b

                @pl.when(t >= SD)
                def _():
                    pltpu.make_async_copy(
                        ones_v, acc_sh.at[dst_v.at[t]], ssem[b]).wait()

                pltpu.async_copy(ones_v, acc_sh.at[dst_v.at[t]], ssem[b],
                                 add=True)

        for b in range(SD):
            pltpu.make_async_copy(
                ones_v, acc_sh.at[dst_v.at[CPT - 1]], ssem[b]).wait()

        plsc.subcore_barrier()
        pltpu.sync_copy(acc_sh.at[sl], out_hbm.at[c].at[sl])

    return sc_deg, sc_agg


def _mm1_body(x_ref, w_ref, o_ref):
    o_ref[...] = lax.dot_general(
        x_ref[...], w_ref[...], (((1,), (0,)), ((), ())),
        preferred_element_type=jnp.float32)


def _scale_body(degp_ref, t1_ref, t1s_ref, dinv_ref):
    deg = degp_ref[0] + degp_ref[1]
    dinv = lax.rsqrt(jnp.maximum(deg, 1e-12))
    dinv_ref[...] = dinv
    t1s_ref[...] = t1_ref[...] * dinv


def _mid_body(aggp_ref, dinv_ref, b1t_ref, t2s_ref):
    agg = aggp_ref[0] + aggp_ref[1]
    dinv = dinv_ref[...]
    h = jnp.maximum(agg * dinv + b1t_ref[...], 0.0)
    t2s_ref[...] = h * dinv


def _out_body(aggp_ref, dinv_ref, w2pk_ref, b2t_ref, summ_ref, o_ref):
    g_pk = (aggp_ref[0] + aggp_ref[1]) * dinv_ref[...]
    z = lax.dot_general(
        g_pk, w2pk_ref[...], (((1,), (0,)), ((), ())),
        preferred_element_type=jnp.float32) + b2t_ref[...]
    b = jnp.max(z, axis=1, keepdims=True)
    e = jnp.exp(z - b)
    gs = lax.dot_general(
        e, summ_ref[...], (((1,), (0,)), ((), ())),
        preferred_element_type=jnp.float32)
    o_ref[...] = z - b - jnp.log(gs)


_mm1 = pl.pallas_call(
    _mm1_body, out_shape=jax.ShapeDtypeStruct((R_PK, 128), jnp.float32))
_scale = pl.pallas_call(
    _scale_body,
    out_shape=(jax.ShapeDtypeStruct((R_PK, 128), jnp.float32),
               jax.ShapeDtypeStruct((R_PK, 128), jnp.float32)))
_mid = pl.pallas_call(
    _mid_body, out_shape=jax.ShapeDtypeStruct((R_PK, 128), jnp.float32))
_out = pl.pallas_call(
    _out_body, out_shape=jax.ShapeDtypeStruct((R_PK, 8 * D_OUT), jnp.float32))


def _pk(a):
    return jnp.reshape(a, (-1, R_PK, 128)) if a.ndim == 3 else (
        jnp.reshape(a, (R_PK, 128)))


def kernel(x, edge_index, W1, b1, W2, b2):
    eidx = jnp.pad(edge_index, ((0, 0), (0, E_PAD - E)),
                   constant_values=PAD_IDX).reshape(2, CHUNKS, CHUNK)
    x_pk = jnp.pad(x, ((0, N_PAD - N), (0, 0))).reshape(R_PK, 8 * D_IN)
    w1_pk = jnp.kron(jnp.eye(8, dtype=jnp.float32), W1.T)
    b1t = jnp.tile(b1, 8).reshape(1, 128)
    zeros_sm = jnp.zeros((ROWS_PT, D_HID), jnp.float32)
    ones_sm = jnp.ones((ROWS_PT, D_HID), jnp.float32)

    sc_deg, sc_agg = _build_sc_kernels()
    degp = sc_deg(ones_sm, eidx, zeros_sm)
    t1_pk = _mm1(x_pk, w1_pk)
    t1s_pk, dinv_pk = _scale(_pk(degp), t1_pk)
    t1s = jnp.reshape(t1s_pk, (N_PAD, D_HID))
    aggp1 = sc_agg(t1s, eidx, zeros_sm)
    t2s_pk = _mid(_pk(aggp1), dinv_pk, b1t)
    t2s = jnp.reshape(t2s_pk, (N_PAD, D_HID))
    aggp2 = sc_agg(t2s, eidx, zeros_sm)
    w2_pk = jnp.kron(jnp.eye(8, dtype=jnp.float32), W2.T)
    b2t = jnp.tile(b2, 8).reshape(1, 8 * D_OUT)
    summ = jnp.kron(jnp.eye(8, dtype=jnp.float32),
                    jnp.ones((D_OUT, D_OUT), jnp.float32))
    out_pk = _out(_pk(aggp2), dinv_pk, w2_pk, b2t, summ)
    return out_pk[:R_N].reshape(N, D_OUT)
